# baseline (speedup 1.0000x reference)
import functools

import jax
import jax.numpy as jnp
import numpy as np
from jax import lax
from jax.experimental import pallas as pl
from jax.experimental.pallas import tpu as pltpu
from jax.experimental.pallas import tpu_sc as plsc

_NC, _NS, _L = 2, 16, 16
_NW = _NC * _NS


def _sc_gather_body(n_vox, n_clust, n_pts,
                    xyp_hbm, zs_hbm, idxt_hbm, oxy_hbm, oz_hbm,
                    sh_xy, sh_z, pxy_v, pz_v, idx_v, oxy_v, oz_v,
                    sem_idx, sem_xy, sem_z, sem_o0, sem_o1):
    clust_per_w = n_clust // _NW
    ngrp = clust_per_w // _L
    wid = lax.axis_index("s") * _NC + lax.axis_index("c")
    c0 = wid * clust_per_w
    sid = lax.axis_index("s")
    idx_cp = pltpu.async_copy(
        idxt_hbm.at[:, pl.ds(c0, clust_per_w)], idx_v, sem_idx)

    seg = n_vox // _NS
    pltpu.sync_copy(xyp_hbm.at[pl.ds(sid * seg, seg)],
                    sh_xy.at[pl.ds(sid * seg, seg)])
    pltpu.sync_copy(zs_hbm.at[pl.ds(sid * seg, seg)],
                    sh_z.at[pl.ds(sid * seg, seg)])
    plsc.subcore_barrier()
    cp_xy = pltpu.async_copy(sh_xy, pxy_v, sem_xy)
    cp_z = pltpu.async_copy(sh_z, pz_v, sem_z)
    cp_xy.wait()
    cp_z.wait()
    idx_cp.wait()

    @plsc.parallel_loop(0, n_pts, step=1, unroll=2)
    def point_body(j):
        for g in range(ngrp):
            idx16 = idx_v[j, pl.ds(g * _L, _L)]
            oxy_v[j, pl.ds(g * _L, _L)] = plsc.load_gather(pxy_v, [idx16])
            oz_v[j, pl.ds(g * _L, _L)] = plsc.load_gather(pz_v, [idx16])

    o0 = pltpu.async_copy(oxy_v, oxy_hbm.at[:, pl.ds(c0, clust_per_w)], sem_o0)
    o1 = pltpu.async_copy(oz_v, oz_hbm.at[:, pl.ds(c0, clust_per_w)], sem_o1)
    o0.wait()
    o1.wait()


_SQRT3_2 = float(np.sqrt(3.0) / 2.0)


def _tc_feats_body(n_pts, xy_ref, z_ref, o_ref):
    ub = lax.bitcast_convert_type(xy_ref[...], jnp.uint32)
    X = lax.bitcast_convert_type(ub << jnp.uint32(16), jnp.float32)
    Y = lax.bitcast_convert_type(ub & jnp.uint32(0xFFFF0000), jnp.float32)
    Z = z_ref[...]
    inv_n = 1.0 / n_pts
    cx = jnp.sum(X, 0, keepdims=True) * inv_n
    cy = jnp.sum(Y, 0, keepdims=True) * inv_n
    cz = jnp.sum(Z, 0, keepdims=True) * inv_n
    Xc, Yc, Zc = X - cx, Y - cy, Z - cz
    axx = jnp.sum(Xc * Xc, 0, keepdims=True)
    axy = jnp.sum(Xc * Yc, 0, keepdims=True)
    axz = jnp.sum(Xc * Zc, 0, keepdims=True)
    ayy = jnp.sum(Yc * Yc, 0, keepdims=True)
    ayz = jnp.sum(Yc * Zc, 0, keepdims=True)
    azz = jnp.sum(Zc * Zc, 0, keepdims=True)

    q = (axx + ayy + azz) * (1.0 / 3.0)
    mxx, myy, mzz = axx - q, ayy - q, azz - q
    p2 = mxx * mxx + myy * myy + mzz * mzz + 2.0 * (axy * axy + axz * axz + ayz * ayz)
    p = jnp.sqrt(p2 * (1.0 / 6.0))
    pd = jnp.maximum(p, 1e-30)
    detM = (mxx * (myy * mzz - ayz * ayz)
            - axy * (axy * mzz - ayz * axz)
            + axz * (axy * ayz - myy * axz))
    rr = jnp.clip(0.5 * detM / (pd * pd * pd), -1.0, 1.0)
    ar = jnp.abs(rr)
    apoly = (1.5707963050 + ar * (-0.2145988016 + ar * (0.0889789874
             + ar * (-0.0501743046 + ar * (0.0308918810 + ar * (-0.0170881256
             + ar * (0.0066700901 + ar * (-0.0012624911))))))))
    acos_pos = jnp.sqrt(jnp.maximum(1.0 - ar, 0.0)) * apoly
    acos_r = jnp.where(rr >= 0.0, acos_pos, float(np.pi) - acos_pos)
    phi = acos_r * (1.0 / 3.0)
    ph2 = phi * phi
    cphi = 1.0 + ph2 * (-0.5 + ph2 * ((1.0 / 24.0) + ph2 * (-(1.0 / 720.0)
           + ph2 * (1.0 / 40320.0))))
    sphi = phi * (1.0 + ph2 * (-(1.0 / 6.0) + ph2 * ((1.0 / 120.0)
           + ph2 * (-(1.0 / 5040.0) + ph2 * (1.0 / 362880.0)))))
    w2 = q + 2.0 * p * cphi
    w0 = q + 2.0 * p * (-0.5 * cphi - _SQRT3_2 * sphi)
    w1 = 3.0 * q - w2 - w0
    dirwt = jnp.where(w2 == 0.0, 0.0, 1.0 - w1 / w2)

    sxx = axx * axx + axy * axy + axz * axz
    sxy = axx * axy + axy * ayy + axz * ayz
    sxz = axx * axz + axy * ayz + axz * azz
    syy = axy * axy + ayy * ayy + ayz * ayz
    syz = axy * axz + ayy * ayz + ayz * azz
    szz = axz * axz + ayz * ayz + azz * azz
    t = w0 + w1
    u = w0 * w1
    P00 = sxx - t * axx + u
    P01 = sxy - t * axy
    P02 = sxz - t * axz
    P11 = syy - t * ayy + u
    P12 = syz - t * ayz
    P22 = szz - t * azz + u
    n0 = P00 * P00 + P01 * P01 + P02 * P02
    n1 = P01 * P01 + P11 * P11 + P12 * P12
    n2 = P02 * P02 + P12 * P12 + P22 * P22
    use0 = (n0 >= n1) & (n0 >= n2)
    use1 = jnp.logical_not(use0) & (n1 >= n2)
    vx = jnp.where(use0, P00, jnp.where(use1, P01, P02))
    vy = jnp.where(use0, P01, jnp.where(use1, P11, P12))
    vz = jnp.where(use0, P02, jnp.where(use1, P12, P22))
    vn = jnp.sqrt(vx * vx + vy * vy + vz * vz)
    inv = jnp.where(vn > 0.0, 1.0 / vn, 0.0)
    vx, vy, vz = vx * inv, vy * inv, vz * inv

    x0 = Xc * vx + Yc * vy + Zc * vz
    xpx = Xc - x0 * vx
    xpy = Yc - x0 * vy
    xpz = Zc - x0 * vz
    np0 = jnp.sqrt(xpx * xpx + xpy * xpy + xpz * xpz)
    sc = jnp.sum(x0 * np0, 0, keepdims=True)
    sgn = jnp.where(sc < 0.0, -1.0, 1.0)
    s = sgn * dirwt
    vfx, vfy, vfz = vx * s, vy * s, vz * s

    iw2 = 1.0 / w2
    size = jnp.full_like(cx, float(n_pts))
    stacked = jnp.concatenate(
        [cx, cy, cz,
         axx * iw2, axy * iw2, axz * iw2,
         axy * iw2, ayy * iw2, ayz * iw2,
         axz * iw2, ayz * iw2, azz * iw2,
         vfx, vfy, vfz, size], axis=0)
    o_ref[...] = stacked.T


def kernel(data, clusts):
    n_vox = data.shape[0]
    n_clust, n_pts = clusts.shape
    vox = data[:, :3].astype(jnp.float32)
    xb = lax.bitcast_convert_type(vox[:, 0].astype(jnp.bfloat16),
                                  jnp.uint16).astype(jnp.uint32)
    yb = lax.bitcast_convert_type(vox[:, 1].astype(jnp.bfloat16),
                                  jnp.uint16).astype(jnp.uint32)
    xyp = lax.bitcast_convert_type(xb | (yb << jnp.uint32(16)), jnp.int32)
    zs = vox[:, 2]
    idxt = clusts.T

    mesh = plsc.VectorSubcoreMesh(core_axis_name="c", subcore_axis_name="s")
    clust_per_w = n_clust // _NW
    oxy, oz = pl.kernel(
        functools.partial(_sc_gather_body, n_vox, n_clust, n_pts),
        out_type=[jax.ShapeDtypeStruct((n_pts, n_clust), jnp.int32),
                  jax.ShapeDtypeStruct((n_pts, n_clust), jnp.float32)],
        mesh=mesh,
        scratch_types=[pltpu.VMEM_SHARED((n_vox,), jnp.int32),
                       pltpu.VMEM_SHARED((n_vox,), jnp.float32),
                       pltpu.VMEM((n_vox,), jnp.int32),
                       pltpu.VMEM((n_vox,), jnp.float32),
                       pltpu.VMEM((n_pts, clust_per_w), jnp.int32),
                       pltpu.VMEM((n_pts, clust_per_w), jnp.int32),
                       pltpu.VMEM((n_pts, clust_per_w), jnp.float32),
                       pltpu.SemaphoreType.DMA,
                       pltpu.SemaphoreType.DMA,
                       pltpu.SemaphoreType.DMA,
                       pltpu.SemaphoreType.DMA,
                       pltpu.SemaphoreType.DMA],
        compiler_params=pltpu.CompilerParams(needs_layout_passes=False),
    )(xyp, zs, idxt)

    bw = 512
    feats = pl.pallas_call(
        functools.partial(_tc_feats_body, n_pts),
        grid=(n_clust // bw,),
        in_specs=[pl.BlockSpec((n_pts, bw), lambda i: (0, i))] * 2,
        out_specs=pl.BlockSpec((bw, 16), lambda i: (i, 0)),
        out_shape=jax.ShapeDtypeStruct((n_clust, 16), jnp.float32),
    )(oxy, oz)
    return feats


# speedup vs baseline: 1.1012x; 1.1012x over previous
import functools

import jax
import jax.numpy as jnp
import numpy as np
from jax import lax
from jax.experimental import pallas as pl
from jax.experimental.pallas import tpu as pltpu
from jax.experimental.pallas import tpu_sc as plsc

_NC, _NS, _L = 2, 16, 16
_NW = _NC * _NS


def _sc_gather_body(n_vox, n_clust, n_pts,
                    xyp_hbm, zs_hbm, idxt_hbm, oxy_hbm, oz_hbm,
                    sh_xy, sh_z, pxy_v, pz_v, idx_v, oxy_v, oz_v,
                    sem_idx, sem_xy, sem_z, sem_o0, sem_o1):
    clust_per_w = n_clust // _NW
    ngrp = clust_per_w // _L
    wid = lax.axis_index("s") * _NC + lax.axis_index("c")
    c0 = wid * clust_per_w
    sid = lax.axis_index("s")
    idx_cp = pltpu.async_copy(
        idxt_hbm.at[:, pl.ds(c0, clust_per_w)], idx_v, sem_idx)

    seg = n_vox // _NS
    pltpu.sync_copy(xyp_hbm.at[pl.ds(sid * seg, seg)],
                    sh_xy.at[pl.ds(sid * seg, seg)])
    pltpu.sync_copy(zs_hbm.at[pl.ds(sid * seg, seg)],
                    sh_z.at[pl.ds(sid * seg, seg)])
    plsc.subcore_barrier()
    cp_xy = pltpu.async_copy(sh_xy, pxy_v, sem_xy)
    cp_z = pltpu.async_copy(sh_z, pz_v, sem_z)
    cp_xy.wait()
    cp_z.wait()
    idx_cp.wait()

    @plsc.parallel_loop(0, n_pts, step=1, unroll=4)
    def point_body(j):
        for g in range(ngrp):
            idx16 = idx_v[j, pl.ds(g * _L, _L)]
            oxy_v[j, pl.ds(g * _L, _L)] = plsc.load_gather(pxy_v, [idx16])
            oz_v[j, pl.ds(g * _L, _L)] = plsc.load_gather(pz_v, [idx16])

    o0 = pltpu.async_copy(oxy_v, oxy_hbm.at[:, pl.ds(c0, clust_per_w)], sem_o0)
    o1 = pltpu.async_copy(oz_v, oz_hbm.at[:, pl.ds(c0, clust_per_w)], sem_o1)
    o0.wait()
    o1.wait()


_SQRT3_2 = float(np.sqrt(3.0) / 2.0)


def _tc_feats_body(n_pts, xy_ref, z_ref, o_ref):
    ub = lax.bitcast_convert_type(xy_ref[...], jnp.uint32)
    X = lax.bitcast_convert_type(ub << jnp.uint32(16), jnp.float32)
    Y = lax.bitcast_convert_type(ub & jnp.uint32(0xFFFF0000), jnp.float32)
    Z = z_ref[...]
    inv_n = 1.0 / n_pts
    cx = jnp.sum(X, 0, keepdims=True) * inv_n
    cy = jnp.sum(Y, 0, keepdims=True) * inv_n
    cz = jnp.sum(Z, 0, keepdims=True) * inv_n
    Xc, Yc, Zc = X - cx, Y - cy, Z - cz
    axx = jnp.sum(Xc * Xc, 0, keepdims=True)
    axy = jnp.sum(Xc * Yc, 0, keepdims=True)
    axz = jnp.sum(Xc * Zc, 0, keepdims=True)
    ayy = jnp.sum(Yc * Yc, 0, keepdims=True)
    ayz = jnp.sum(Yc * Zc, 0, keepdims=True)
    azz = jnp.sum(Zc * Zc, 0, keepdims=True)

    q = (axx + ayy + azz) * (1.0 / 3.0)
    mxx, myy, mzz = axx - q, ayy - q, azz - q
    p2 = mxx * mxx + myy * myy + mzz * mzz + 2.0 * (axy * axy + axz * axz + ayz * ayz)
    p = jnp.sqrt(p2 * (1.0 / 6.0))
    pd = jnp.maximum(p, 1e-30)
    detM = (mxx * (myy * mzz - ayz * ayz)
            - axy * (axy * mzz - ayz * axz)
            + axz * (axy * ayz - myy * axz))
    rr = jnp.clip(0.5 * detM / (pd * pd * pd), -1.0, 1.0)
    ar = jnp.abs(rr)
    apoly = (1.5707963050 + ar * (-0.2145988016 + ar * (0.0889789874
             + ar * (-0.0501743046 + ar * (0.0308918810 + ar * (-0.0170881256
             + ar * (0.0066700901 + ar * (-0.0012624911))))))))
    acos_pos = jnp.sqrt(jnp.maximum(1.0 - ar, 0.0)) * apoly
    acos_r = jnp.where(rr >= 0.0, acos_pos, float(np.pi) - acos_pos)
    phi = acos_r * (1.0 / 3.0)
    ph2 = phi * phi
    cphi = 1.0 + ph2 * (-0.5 + ph2 * ((1.0 / 24.0) + ph2 * (-(1.0 / 720.0)
           + ph2 * (1.0 / 40320.0))))
    sphi = phi * (1.0 + ph2 * (-(1.0 / 6.0) + ph2 * ((1.0 / 120.0)
           + ph2 * (-(1.0 / 5040.0) + ph2 * (1.0 / 362880.0)))))
    w2 = q + 2.0 * p * cphi
    w0 = q + 2.0 * p * (-0.5 * cphi - _SQRT3_2 * sphi)
    w1 = 3.0 * q - w2 - w0
    dirwt = jnp.where(w2 == 0.0, 0.0, 1.0 - w1 / w2)

    sxx = axx * axx + axy * axy + axz * axz
    sxy = axx * axy + axy * ayy + axz * ayz
    sxz = axx * axz + axy * ayz + axz * azz
    syy = axy * axy + ayy * ayy + ayz * ayz
    syz = axy * axz + ayy * ayz + ayz * azz
    szz = axz * axz + ayz * ayz + azz * azz
    t = w0 + w1
    u = w0 * w1
    P00 = sxx - t * axx + u
    P01 = sxy - t * axy
    P02 = sxz - t * axz
    P11 = syy - t * ayy + u
    P12 = syz - t * ayz
    P22 = szz - t * azz + u
    n0 = P00 * P00 + P01 * P01 + P02 * P02
    n1 = P01 * P01 + P11 * P11 + P12 * P12
    n2 = P02 * P02 + P12 * P12 + P22 * P22
    use0 = (n0 >= n1) & (n0 >= n2)
    use1 = jnp.logical_not(use0) & (n1 >= n2)
    vx = jnp.where(use0, P00, jnp.where(use1, P01, P02))
    vy = jnp.where(use0, P01, jnp.where(use1, P11, P12))
    vz = jnp.where(use0, P02, jnp.where(use1, P12, P22))
    vn = jnp.sqrt(vx * vx + vy * vy + vz * vz)
    inv = jnp.where(vn > 0.0, 1.0 / vn, 0.0)
    vx, vy, vz = vx * inv, vy * inv, vz * inv

    x0 = Xc * vx + Yc * vy + Zc * vz
    xpx = Xc - x0 * vx
    xpy = Yc - x0 * vy
    xpz = Zc - x0 * vz
    np0 = jnp.sqrt(xpx * xpx + xpy * xpy + xpz * xpz)
    sc = jnp.sum(x0 * np0, 0, keepdims=True)
    sgn = jnp.where(sc < 0.0, -1.0, 1.0)
    s = sgn * dirwt
    vfx, vfy, vfz = vx * s, vy * s, vz * s

    iw2 = 1.0 / w2
    size = jnp.full_like(cx, float(n_pts))
    o_ref[...] = jnp.concatenate(
        [cx, cy, cz,
         axx * iw2, axy * iw2, axz * iw2,
         axy * iw2, ayy * iw2, ayz * iw2,
         axz * iw2, ayz * iw2, azz * iw2,
         vfx, vfy, vfz, size], axis=0)


def kernel(data, clusts):
    n_vox = data.shape[0]
    n_clust, n_pts = clusts.shape
    vox = data[:, :3].astype(jnp.float32)
    xb = lax.bitcast_convert_type(vox[:, 0].astype(jnp.bfloat16),
                                  jnp.uint16).astype(jnp.uint32)
    yb = lax.bitcast_convert_type(vox[:, 1].astype(jnp.bfloat16),
                                  jnp.uint16).astype(jnp.uint32)
    xyp = lax.bitcast_convert_type(xb | (yb << jnp.uint32(16)), jnp.int32)
    zs = vox[:, 2]
    idxt = clusts.T

    mesh = plsc.VectorSubcoreMesh(core_axis_name="c", subcore_axis_name="s")
    clust_per_w = n_clust // _NW
    oxy, oz = pl.kernel(
        functools.partial(_sc_gather_body, n_vox, n_clust, n_pts),
        out_type=[jax.ShapeDtypeStruct((n_pts, n_clust), jnp.int32),
                  jax.ShapeDtypeStruct((n_pts, n_clust), jnp.float32)],
        mesh=mesh,
        scratch_types=[pltpu.VMEM_SHARED((n_vox,), jnp.int32),
                       pltpu.VMEM_SHARED((n_vox,), jnp.float32),
                       pltpu.VMEM((n_vox,), jnp.int32),
                       pltpu.VMEM((n_vox,), jnp.float32),
                       pltpu.VMEM((n_pts, clust_per_w), jnp.int32),
                       pltpu.VMEM((n_pts, clust_per_w), jnp.int32),
                       pltpu.VMEM((n_pts, clust_per_w), jnp.float32),
                       pltpu.SemaphoreType.DMA,
                       pltpu.SemaphoreType.DMA,
                       pltpu.SemaphoreType.DMA,
                       pltpu.SemaphoreType.DMA,
                       pltpu.SemaphoreType.DMA],
        compiler_params=pltpu.CompilerParams(needs_layout_passes=False),
    )(xyp, zs, idxt)

    bw = 512
    feats_t = pl.pallas_call(
        functools.partial(_tc_feats_body, n_pts),
        grid=(n_clust // bw,),
        in_specs=[pl.BlockSpec((n_pts, bw), lambda i: (0, i))] * 2,
        out_specs=pl.BlockSpec((16, bw), lambda i: (0, i)),
        out_shape=jax.ShapeDtypeStruct((16, n_clust), jnp.float32),
    )(oxy, oz)
    return feats_t.T


# speedup vs baseline: 1.1094x; 1.0074x over previous
import functools

import jax
import jax.numpy as jnp
import numpy as np
from jax import lax
from jax.experimental import pallas as pl
from jax.experimental.pallas import tpu as pltpu
from jax.experimental.pallas import tpu_sc as plsc

_NC, _NS, _L = 2, 16, 16
_NW = _NC * _NS


def _sc_gather_body(n_vox, n_clust, n_pts,
                    xyp_hbm, zs_hbm, idxt_hbm, oxy_hbm, oz_hbm,
                    sh_xy, sh_z, pxy_v, pz_v, idx_v, oxy_v, oz_v,
                    sem_idx, sem_xy, sem_z, sem_o0, sem_o1):
    clust_per_w = n_clust // _NW
    ngrp = clust_per_w // _L
    wid = lax.axis_index("s") * _NC + lax.axis_index("c")
    c0 = wid * clust_per_w
    sid = lax.axis_index("s")
    idx_cp = pltpu.async_copy(
        idxt_hbm.at[:, pl.ds(c0, clust_per_w)], idx_v, sem_idx)

    seg = n_vox // _NS
    pltpu.sync_copy(xyp_hbm.at[pl.ds(sid * seg, seg)],
                    sh_xy.at[pl.ds(sid * seg, seg)])
    pltpu.sync_copy(zs_hbm.at[pl.ds(sid * seg, seg)],
                    sh_z.at[pl.ds(sid * seg, seg)])
    plsc.subcore_barrier()
    cp_xy = pltpu.async_copy(sh_xy, pxy_v, sem_xy)
    cp_z = pltpu.async_copy(sh_z, pz_v, sem_z)
    cp_xy.wait()
    cp_z.wait()
    idx_cp.wait()

    nh = n_pts // 2
    out_cps = []
    for h in range(2):
        @plsc.parallel_loop(h * nh, (h + 1) * nh, step=1, unroll=4)
        def point_body(j):
            for g in range(ngrp):
                idx16 = idx_v[j, pl.ds(g * _L, _L)]
                oxy_v[j, pl.ds(g * _L, _L)] = plsc.load_gather(pxy_v, [idx16])
                oz_v[j, pl.ds(g * _L, _L)] = plsc.load_gather(pz_v, [idx16])

        out_cps.append(pltpu.async_copy(
            oxy_v.at[pl.ds(h * nh, nh)],
            oxy_hbm.at[pl.ds(h * nh, nh), pl.ds(c0, clust_per_w)], sem_o0))
        out_cps.append(pltpu.async_copy(
            oz_v.at[pl.ds(h * nh, nh)],
            oz_hbm.at[pl.ds(h * nh, nh), pl.ds(c0, clust_per_w)], sem_o1))

    for cp in out_cps:
        cp.wait()


_SQRT3_2 = float(np.sqrt(3.0) / 2.0)


def _tc_feats_body(n_pts, xy_ref, z_ref, o_ref):
    ub = lax.bitcast_convert_type(xy_ref[...], jnp.uint32)
    X = lax.bitcast_convert_type(ub << jnp.uint32(16), jnp.float32)
    Y = lax.bitcast_convert_type(ub & jnp.uint32(0xFFFF0000), jnp.float32)
    Z = z_ref[...]
    inv_n = 1.0 / n_pts
    cx = jnp.sum(X, 0, keepdims=True) * inv_n
    cy = jnp.sum(Y, 0, keepdims=True) * inv_n
    cz = jnp.sum(Z, 0, keepdims=True) * inv_n
    Xc, Yc, Zc = X - cx, Y - cy, Z - cz
    axx = jnp.sum(Xc * Xc, 0, keepdims=True)
    axy = jnp.sum(Xc * Yc, 0, keepdims=True)
    axz = jnp.sum(Xc * Zc, 0, keepdims=True)
    ayy = jnp.sum(Yc * Yc, 0, keepdims=True)
    ayz = jnp.sum(Yc * Zc, 0, keepdims=True)
    azz = jnp.sum(Zc * Zc, 0, keepdims=True)

    q = (axx + ayy + azz) * (1.0 / 3.0)
    mxx, myy, mzz = axx - q, ayy - q, azz - q
    p2 = mxx * mxx + myy * myy + mzz * mzz + 2.0 * (axy * axy + axz * axz + ayz * ayz)
    p = jnp.sqrt(p2 * (1.0 / 6.0))
    pd = jnp.maximum(p, 1e-30)
    detM = (mxx * (myy * mzz - ayz * ayz)
            - axy * (axy * mzz - ayz * axz)
            + axz * (axy * ayz - myy * axz))
    rr = jnp.clip(0.5 * detM / (pd * pd * pd), -1.0, 1.0)
    ar = jnp.abs(rr)
    apoly = (1.5707963050 + ar * (-0.2145988016 + ar * (0.0889789874
             + ar * (-0.0501743046 + ar * (0.0308918810 + ar * (-0.0170881256
             + ar * (0.0066700901 + ar * (-0.0012624911))))))))
    acos_pos = jnp.sqrt(jnp.maximum(1.0 - ar, 0.0)) * apoly
    acos_r = jnp.where(rr >= 0.0, acos_pos, float(np.pi) - acos_pos)
    phi = acos_r * (1.0 / 3.0)
    ph2 = phi * phi
    cphi = 1.0 + ph2 * (-0.5 + ph2 * ((1.0 / 24.0) + ph2 * (-(1.0 / 720.0)
           + ph2 * (1.0 / 40320.0))))
    sphi = phi * (1.0 + ph2 * (-(1.0 / 6.0) + ph2 * ((1.0 / 120.0)
           + ph2 * (-(1.0 / 5040.0) + ph2 * (1.0 / 362880.0)))))
    w2 = q + 2.0 * p * cphi
    w0 = q + 2.0 * p * (-0.5 * cphi - _SQRT3_2 * sphi)
    w1 = 3.0 * q - w2 - w0
    dirwt = jnp.where(w2 == 0.0, 0.0, 1.0 - w1 / w2)

    sxx = axx * axx + axy * axy + axz * axz
    sxy = axx * axy + axy * ayy + axz * ayz
    sxz = axx * axz + axy * ayz + axz * azz
    syy = axy * axy + ayy * ayy + ayz * ayz
    syz = axy * axz + ayy * ayz + ayz * azz
    szz = axz * axz + ayz * ayz + azz * azz
    t = w0 + w1
    u = w0 * w1
    P00 = sxx - t * axx + u
    P01 = sxy - t * axy
    P02 = sxz - t * axz
    P11 = syy - t * ayy + u
    P12 = syz - t * ayz
    P22 = szz - t * azz + u
    n0 = P00 * P00 + P01 * P01 + P02 * P02
    n1 = P01 * P01 + P11 * P11 + P12 * P12
    n2 = P02 * P02 + P12 * P12 + P22 * P22
    use0 = (n0 >= n1) & (n0 >= n2)
    use1 = jnp.logical_not(use0) & (n1 >= n2)
    vx = jnp.where(use0, P00, jnp.where(use1, P01, P02))
    vy = jnp.where(use0, P01, jnp.where(use1, P11, P12))
    vz = jnp.where(use0, P02, jnp.where(use1, P12, P22))
    vn = jnp.sqrt(vx * vx + vy * vy + vz * vz)
    inv = jnp.where(vn > 0.0, 1.0 / vn, 0.0)
    vx, vy, vz = vx * inv, vy * inv, vz * inv

    x0 = Xc * vx + Yc * vy + Zc * vz
    xpx = Xc - x0 * vx
    xpy = Yc - x0 * vy
    xpz = Zc - x0 * vz
    np0 = jnp.sqrt(xpx * xpx + xpy * xpy + xpz * xpz)
    sc = jnp.sum(x0 * np0, 0, keepdims=True)
    sgn = jnp.where(sc < 0.0, -1.0, 1.0)
    s = sgn * dirwt
    vfx, vfy, vfz = vx * s, vy * s, vz * s

    iw2 = 1.0 / w2
    size = jnp.full_like(cx, float(n_pts))
    o_ref[...] = jnp.concatenate(
        [cx, cy, cz,
         axx * iw2, axy * iw2, axz * iw2,
         axy * iw2, ayy * iw2, ayz * iw2,
         axz * iw2, ayz * iw2, azz * iw2,
         vfx, vfy, vfz, size], axis=0)


def kernel(data, clusts):
    n_vox = data.shape[0]
    n_clust, n_pts = clusts.shape
    vox = data[:, :3].astype(jnp.float32)
    xb = lax.bitcast_convert_type(vox[:, 0].astype(jnp.bfloat16),
                                  jnp.uint16).astype(jnp.uint32)
    yb = lax.bitcast_convert_type(vox[:, 1].astype(jnp.bfloat16),
                                  jnp.uint16).astype(jnp.uint32)
    xyp = lax.bitcast_convert_type(xb | (yb << jnp.uint32(16)), jnp.int32)
    zs = vox[:, 2]
    idxt = clusts.T

    mesh = plsc.VectorSubcoreMesh(core_axis_name="c", subcore_axis_name="s")
    clust_per_w = n_clust // _NW
    oxy, oz = pl.kernel(
        functools.partial(_sc_gather_body, n_vox, n_clust, n_pts),
        out_type=[jax.ShapeDtypeStruct((n_pts, n_clust), jnp.int32),
                  jax.ShapeDtypeStruct((n_pts, n_clust), jnp.float32)],
        mesh=mesh,
        scratch_types=[pltpu.VMEM_SHARED((n_vox,), jnp.int32),
                       pltpu.VMEM_SHARED((n_vox,), jnp.float32),
                       pltpu.VMEM((n_vox,), jnp.int32),
                       pltpu.VMEM((n_vox,), jnp.float32),
                       pltpu.VMEM((n_pts, clust_per_w), jnp.int32),
                       pltpu.VMEM((n_pts, clust_per_w), jnp.int32),
                       pltpu.VMEM((n_pts, clust_per_w), jnp.float32),
                       pltpu.SemaphoreType.DMA,
                       pltpu.SemaphoreType.DMA,
                       pltpu.SemaphoreType.DMA,
                       pltpu.SemaphoreType.DMA,
                       pltpu.SemaphoreType.DMA],
        compiler_params=pltpu.CompilerParams(needs_layout_passes=False),
    )(xyp, zs, idxt)

    bw = 512
    feats_t = pl.pallas_call(
        functools.partial(_tc_feats_body, n_pts),
        grid=(n_clust // bw,),
        in_specs=[pl.BlockSpec((n_pts, bw), lambda i: (0, i))] * 2,
        out_specs=pl.BlockSpec((16, bw), lambda i: (0, i)),
        out_shape=jax.ShapeDtypeStruct((16, n_clust), jnp.float32),
    )(oxy, oz)
    return feats_t.T
